# baseline (speedup 1.0000x reference)
import functools

import jax
import jax.numpy as jnp
from jax import lax
from jax.experimental import pallas as pl
from jax.experimental.pallas import tpu as pltpu
from jax.experimental.pallas import tpu_sc as plsc

B, C, H, W = 64, 96, 64, 64
SP = H * W
W_L2 = 0.6 / float(B * C * H * W)
W_KL = 0.4 / float(B)
LN2 = 0.6931471805599453

_INFO = plsc.get_sparse_core_info()
NC, NS, L = _INFO.num_cores, _INFO.num_subcores, _INFO.num_lanes
NW = NC * NS
BPW = B // NW
CG = 8
C0 = 80
CB = 16
NCH = SP // 16

_LOG2C = (
    1.442659260e00,
    -7.204583953e-01,
    4.728939106e-01,
    -3.242965810e-01,
    1.923098910e-01,
    -7.835914090e-02,
    1.525174349e-02,
)


def _log2_mant(t):
    p = jnp.float32(_LOG2C[6])
    for c in _LOG2C[5::-1]:
        p = p * t + jnp.float32(c)
    return p * t


def _terms(av, xv, accs):
    l2a, kla = accs
    abits = lax.bitcast_convert_type(av, jnp.int32)
    xbits = lax.bitcast_convert_type(xv, jnp.int32)
    de = (abits >> 23) - (xbits >> 23)
    am = (
        lax.bitcast_convert_type((abits & 0x7FFFFF) | 0x3F800000, jnp.float32)
        - 1.0
    )
    xm = (
        lax.bitcast_convert_type((xbits & 0x7FFFFF) | 0x3F800000, jnp.float32)
        - 1.0
    )
    d = de.astype(jnp.float32) + (_log2_mant(am) - _log2_mant(xm))
    kla = kla + av * d
    df = av - xv
    return (l2a + df * df, kla)


def _lperm(v, idx):
    gd = lax.GatherDimensionNumbers(
        offset_dims=(), collapsed_slice_dims=(0,), start_index_map=(0,)
    )
    return lax.gather(
        v, idx[:, None], gd, (1,), mode=lax.GatherScatterMode.PROMISE_IN_BOUNDS
    )


def _transpose16(regs):
    lanes = lax.iota(jnp.int32, L)
    regs = list(regs)
    for k in (8, 4, 2, 1):
        mask = (lanes & k) == 0
        rotp = (lanes - k) & (L - 1)
        rotm = (lanes + k) & (L - 1)
        for i in range(L):
            if i & k:
                continue
            j = i | k
            ai, aj = regs[i], regs[j]
            regs[i] = jnp.where(mask, ai, _lperm(aj, rotp))
            regs[j] = jnp.where(mask, _lperm(ai, rotm), aj)
    return regs


def _sc_body(hp_hbm, xp_hbm, lbl_hbm, out_hbm, a_v, x_v, l_v, o_v):
    wid = lax.axis_index("s") * NC + lax.axis_index("c")
    pltpu.sync_copy(lbl_hbm.at[wid], l_v)

    def chunks_fwd(_, accs):
        def step(i, accs):
            base = pl.multiple_of(i * L, L)
            return _terms(a_v[pl.ds(base, L)], x_v[pl.ds(base, L)], accs)

        return lax.fori_loop(0, CG * NCH, step, accs, unroll=8)

    def chunks_rev(_, accs):
        def step(i, accs):
            cc = i // NCH
            ii = i % NCH
            sbase = pl.multiple_of(cc * SP + (NCH - 1 - ii) * L, L)
            av = lax.rev(a_v[pl.ds(sbase, L)], (0,))
            base = pl.multiple_of(i * L, L)
            return _terms(av, x_v[pl.ds(base, L)], accs)

        return lax.fori_loop(0, CG * NCH, step, accs, unroll=8)

    def blocks_t(flip_lanes):
        nb = (H // L) * (W // L)

        def body(_, accs):
            def blk(ij, accs):
                cc = ij // nb
                k = ij % nb
                cb = cc * SP
                bi = k // (W // L)
                bj = k % (W // L)
                if flip_lanes:
                    r0 = (W // L - 1 - bj) * L * W
                    c0 = bi * L
                else:
                    r0 = bj * L * W
                    c0 = (W // L - 1 - bi) * L
                regs = [
                    a_v[pl.ds(pl.multiple_of(cb + r0 + u * W + c0, L), L)]
                    for u in range(L)
                ]
                t = _transpose16(regs)
                for i2 in range(L):
                    if flip_lanes:
                        av = lax.rev(t[i2], (0,))
                    else:
                        av = t[L - 1 - i2]
                    xb = pl.multiple_of(cb + (bi * L + i2) * W + bj * L, L)
                    accs = _terms(av, x_v[pl.ds(xb, L)], accs)
                return accs

            return lax.fori_loop(0, CG * nb, blk, accs)

        return body

    paths = [chunks_fwd, blocks_t(True), chunks_rev, blocks_t(False)]

    z = jnp.zeros((L,), jnp.float32)
    o_v[pl.ds(0, L)] = z
    o_v[pl.ds(L, L)] = z

    lvec = l_v[pl.ds(0, L)]

    def batch_body(bi, r):
        b = wid * BPW + bi

        def chan_loop(c, _):
            off = pl.multiple_of(C0 * SP + c * CG * SP, CG * SP)
            pltpu.sync_copy(hp_hbm.at[b, pl.ds(off, CG * SP)], a_v)
            pltpu.sync_copy(xp_hbm.at[b, pl.ds(off, CG * SP)], x_v)
            for rr in range(4):

                @pl.when(r == rr)
                def _run_path(rr=rr):
                    accs = (o_v[pl.ds(0, L)], o_v[pl.ds(L, L)])
                    l2a, kla = paths[rr](0, accs)
                    o_v[pl.ds(0, L)] = l2a
                    o_v[pl.ds(L, L)] = kla

            return 0

        lax.fori_loop(0, (C - C0) // CG, chan_loop, 0)

    for bi in range(BPW):
        batch_body(bi, lvec[bi])
    o_v[pl.ds(L, L)] = o_v[pl.ds(L, L)] * jnp.float32(LN2)
    pltpu.sync_copy(o_v, out_hbm.at[wid])


def _tc_body(label_ref, hp_ref, hprot_ref, out_ref, acc_ref):
    b = pl.program_id(0)
    c = pl.program_id(1)

    @pl.when((b == 0) & (c == 0))
    def _init():
        acc_ref[...] = jnp.zeros_like(acc_ref)

    r = label_ref[b]
    a = hp_ref[0]
    x = hprot_ref[0]

    la = jnp.log(a)
    lx = jnp.log(jnp.maximum(x, 1e-9))
    d = (2.0 * W_L2) * x + W_KL * lx

    row = jax.lax.broadcasted_iota(jnp.int32, (W, W), 0)
    col = jax.lax.broadcasted_iota(jnp.int32, (W, W), 1)
    jmat = (col == (W - 1) - row).astype(jnp.float32)

    def _t(v):
        return jnp.swapaxes(v, 1, 2)

    def _r(v):
        flat = v.reshape(CB * H, W)
        return jax.lax.dot(
            flat, jmat, precision=jax.lax.Precision.DEFAULT
        ).reshape(CB, H, W)

    p = jax.lax.switch(
        r,
        [
            lambda v: v,
            lambda v: _t(_r(v)),
            lambda v: _r(_t(_r(_t(v)))),
            lambda v: _r(_t(v)),
        ],
        d,
    )
    term = W_L2 * (a * a + x * x) + W_KL * (a * la) - a * p
    acc_ref[...] += jnp.sum(term, axis=0)

    @pl.when((b == B - 1) & (c == C0 // CB - 1))
    def _fin():
        out_ref[0, 0] = jnp.sum(acc_ref[...])


def _tc_loss(labels, hp, hp_rot):
    grid_spec = pltpu.PrefetchScalarGridSpec(
        num_scalar_prefetch=1,
        grid=(B, C0 // CB),
        in_specs=[
            pl.BlockSpec((1, CB, H, W), lambda b, c, L: (b, c, 0, 0)),
            pl.BlockSpec((1, CB, H, W), lambda b, c, L: (b, c, 0, 0)),
        ],
        out_specs=pl.BlockSpec(
            (1, 1), lambda b, c, L: (0, 0), memory_space=pltpu.SMEM
        ),
        scratch_shapes=[pltpu.VMEM((H, W), jnp.float32)],
    )
    out = pl.pallas_call(
        _tc_body,
        grid_spec=grid_spec,
        out_shape=jax.ShapeDtypeStruct((1, 1), jnp.float32),
    )(labels, hp, hp_rot)
    return out[0, 0]


def _sc_loss(hp, hp_rot, label_rot):
    run = pl.kernel(
        _sc_body,
        out_type=jax.ShapeDtypeStruct((NW, 2 * L), jnp.float32),
        mesh=plsc.VectorSubcoreMesh(core_axis_name="c", subcore_axis_name="s"),
        scratch_types=[
            pltpu.VMEM((CG * SP,), jnp.float32),
            pltpu.VMEM((CG * SP,), jnp.float32),
            pltpu.VMEM((L,), jnp.int32),
            pltpu.VMEM((2 * L,), jnp.float32),
        ],
    )
    lbl = jnp.zeros((NW, L), jnp.int32).at[:, :BPW].set(label_rot.reshape(NW, BPW))
    parts = run(hp.reshape(B, C * SP), hp_rot.reshape(B, C * SP), lbl)
    return W_L2 * jnp.sum(parts[:, :L]) + W_KL * jnp.sum(parts[:, L:])


@jax.jit
def _loss(hp, hp_rot, labels):
    return _tc_loss(labels, hp, hp_rot) + _sc_loss(hp, hp_rot, labels)


def kernel(hp, hp_rot, label_rot):
    return _loss(hp, hp_rot, label_rot.astype(jnp.int32))


# speedup vs baseline: 1.3978x; 1.3978x over previous
import functools

import jax
import jax.numpy as jnp
from jax import lax
from jax.experimental import pallas as pl
from jax.experimental.pallas import tpu as pltpu
from jax.experimental.pallas import tpu_sc as plsc

B, C, H, W = 64, 96, 64, 64
SP = H * W
W_L2 = 0.6 / float(B * C * H * W)
W_KL = 0.4 / float(B)
LN2 = 0.6931471805599453

_INFO = plsc.get_sparse_core_info()
NC, NS, L = _INFO.num_cores, _INFO.num_subcores, _INFO.num_lanes
NW = NC * NS
BPW = B // NW
CG = 4
C0 = 80
CB = 16
NCH = SP // 16

_LOG2C = (
    1.442659260e00,
    -7.204583953e-01,
    4.728939106e-01,
    -3.242965810e-01,
    1.923098910e-01,
    -7.835914090e-02,
    1.525174349e-02,
)


def _log2_mant(t):
    p = jnp.float32(_LOG2C[6])
    for c in _LOG2C[5::-1]:
        p = p * t + jnp.float32(c)
    return p * t


def _terms(av, xv, accs):
    l2a, kla = accs
    abits = lax.bitcast_convert_type(av, jnp.int32)
    xbits = lax.bitcast_convert_type(xv, jnp.int32)
    de = (abits >> 23) - (xbits >> 23)
    am = (
        lax.bitcast_convert_type((abits & 0x7FFFFF) | 0x3F800000, jnp.float32)
        - 1.0
    )
    xm = (
        lax.bitcast_convert_type((xbits & 0x7FFFFF) | 0x3F800000, jnp.float32)
        - 1.0
    )
    d = de.astype(jnp.float32) + (_log2_mant(am) - _log2_mant(xm))
    kla = kla + av * d
    df = av - xv
    return (l2a + df * df, kla)


def _lperm(v, idx):
    gd = lax.GatherDimensionNumbers(
        offset_dims=(), collapsed_slice_dims=(0,), start_index_map=(0,)
    )
    return lax.gather(
        v, idx[:, None], gd, (1,), mode=lax.GatherScatterMode.PROMISE_IN_BOUNDS
    )


def _transpose16(regs):
    lanes = lax.iota(jnp.int32, L)
    regs = list(regs)
    for k in (8, 4, 2, 1):
        mask = (lanes & k) == 0
        rotp = (lanes - k) & (L - 1)
        rotm = (lanes + k) & (L - 1)
        for i in range(L):
            if i & k:
                continue
            j = i | k
            ai, aj = regs[i], regs[j]
            regs[i] = jnp.where(mask, ai, _lperm(aj, rotp))
            regs[j] = jnp.where(mask, _lperm(ai, rotm), aj)
    return regs


def _sc_body(hp_hbm, xp_hbm, lbl_hbm, out_hbm, a_v, x_v, l_v, o_v):
    wid = lax.axis_index("s") * NC + lax.axis_index("c")
    pltpu.sync_copy(lbl_hbm.at[wid], l_v)

    WCH = W // L

    def chunks_fwd(_, accs):
        def step(i, accs):
            cc = i // NCH
            rem = i % NCH
            row = rem // WCH
            c4 = rem % WCH
            base = pl.multiple_of(c4 * L, L)
            return _terms(
                a_v[cc, row, pl.ds(base, L)],
                x_v[cc, row, pl.ds(base, L)],
                accs,
            )

        return lax.fori_loop(0, CG * NCH, step, accs, unroll=8)

    def chunks_rev(_, accs):
        def step(i, accs):
            cc = i // NCH
            rem = i % NCH
            row = rem // WCH
            c4 = rem % WCH
            sbase = pl.multiple_of((WCH - 1 - c4) * L, L)
            av = lax.rev(a_v[cc, H - 1 - row, pl.ds(sbase, L)], (0,))
            base = pl.multiple_of(c4 * L, L)
            return _terms(av, x_v[cc, row, pl.ds(base, L)], accs)

        return lax.fori_loop(0, CG * NCH, step, accs, unroll=8)

    def blocks_t(flip_lanes):
        nb = (H // L) * (W // L)

        def body(_, accs):
            def blk(ij, accs):
                cc = ij // nb
                k = ij % nb
                bi = k // (W // L)
                bj = k % (W // L)
                if flip_lanes:
                    r0 = (W // L - 1 - bj) * L
                    c0 = bi * L
                else:
                    r0 = bj * L
                    c0 = (W // L - 1 - bi) * L
                regs = [
                    a_v[cc, r0 + u, pl.ds(pl.multiple_of(c0, L), L)]
                    for u in range(L)
                ]
                t = _transpose16(regs)
                for i2 in range(L):
                    if flip_lanes:
                        av = lax.rev(t[i2], (0,))
                    else:
                        av = t[L - 1 - i2]
                    accs = _terms(
                        av,
                        x_v[cc, bi * L + i2, pl.ds(pl.multiple_of(bj * L, L), L)],
                        accs,
                    )
                return accs

            return lax.fori_loop(0, CG * nb, blk, accs)

        return body

    paths = [chunks_fwd, blocks_t(True), chunks_rev, blocks_t(False)]

    z = jnp.zeros((L,), jnp.float32)
    o_v[pl.ds(0, L)] = z
    o_v[pl.ds(L, L)] = z

    lvec = l_v[pl.ds(0, L)]

    def batch_body(bi, r):
        b = wid * BPW + bi

        def chan_loop(c, _):
            off = pl.multiple_of(C0 + c * CG, CG)
            pltpu.sync_copy(hp_hbm.at[b, pl.ds(off, CG)], a_v)
            pltpu.sync_copy(xp_hbm.at[b, pl.ds(off, CG)], x_v)
            for rr in range(4):

                @pl.when(r == rr)
                def _run_path(rr=rr):
                    accs = (o_v[pl.ds(0, L)], o_v[pl.ds(L, L)])
                    l2a, kla = paths[rr](0, accs)
                    o_v[pl.ds(0, L)] = l2a
                    o_v[pl.ds(L, L)] = kla

            return 0

        lax.fori_loop(0, (C - C0) // CG, chan_loop, 0)

    for bi in range(BPW):
        batch_body(bi, lvec[bi])
    o_v[pl.ds(L, L)] = o_v[pl.ds(L, L)] * jnp.float32(LN2)
    pltpu.sync_copy(o_v, out_hbm.at[wid])


def _tc_body(label_ref, hp_ref, hprot_ref, out_ref, acc_ref):
    b = pl.program_id(0)
    c = pl.program_id(1)

    @pl.when((b == 0) & (c == 0))
    def _init():
        acc_ref[...] = jnp.zeros_like(acc_ref)

    r = label_ref[b]
    a = hp_ref[0]
    x = hprot_ref[0]

    la = jnp.log(a)
    lx = jnp.log(jnp.maximum(x, 1e-9))
    d = (2.0 * W_L2) * x + W_KL * lx

    row = jax.lax.broadcasted_iota(jnp.int32, (W, W), 0)
    col = jax.lax.broadcasted_iota(jnp.int32, (W, W), 1)
    jmat = (col == (W - 1) - row).astype(jnp.float32)

    def _t(v):
        return jnp.swapaxes(v, 1, 2)

    def _r(v):
        flat = v.reshape(CB * H, W)
        return jax.lax.dot(
            flat, jmat, precision=jax.lax.Precision.DEFAULT
        ).reshape(CB, H, W)

    p = jax.lax.switch(
        r,
        [
            lambda v: v,
            lambda v: _t(_r(v)),
            lambda v: _r(_t(_r(_t(v)))),
            lambda v: _r(_t(v)),
        ],
        d,
    )
    term = W_L2 * (a * a + x * x) + W_KL * (a * la) - a * p
    acc_ref[...] += jnp.sum(term, axis=0)

    @pl.when((b == B - 1) & (c == C0 // CB - 1))
    def _fin():
        out_ref[0, 0] = jnp.sum(acc_ref[...])


def _tc_loss(labels, hp, hp_rot):
    grid_spec = pltpu.PrefetchScalarGridSpec(
        num_scalar_prefetch=1,
        grid=(B, C0 // CB),
        in_specs=[
            pl.BlockSpec((1, CB, H, W), lambda b, c, L: (b, c, 0, 0)),
            pl.BlockSpec((1, CB, H, W), lambda b, c, L: (b, c, 0, 0)),
        ],
        out_specs=pl.BlockSpec(
            (1, 1), lambda b, c, L: (0, 0), memory_space=pltpu.SMEM
        ),
        scratch_shapes=[pltpu.VMEM((H, W), jnp.float32)],
    )
    out = pl.pallas_call(
        _tc_body,
        grid_spec=grid_spec,
        out_shape=jax.ShapeDtypeStruct((1, 1), jnp.float32),
    )(labels, hp, hp_rot)
    return out[0, 0]


def _sc_loss(hp, hp_rot, label_rot):
    run = pl.kernel(
        _sc_body,
        out_type=jax.ShapeDtypeStruct((NW, 2 * L), jnp.float32),
        mesh=plsc.VectorSubcoreMesh(core_axis_name="c", subcore_axis_name="s"),
        scratch_types=[
            pltpu.VMEM((CG, H, W), jnp.float32),
            pltpu.VMEM((CG, H, W), jnp.float32),
            pltpu.VMEM((L,), jnp.int32),
            pltpu.VMEM((2 * L,), jnp.float32),
        ],
    )
    lbl = jnp.zeros((NW, L), jnp.int32).at[:, :BPW].set(label_rot.reshape(NW, BPW))
    parts = run(hp, hp_rot, lbl)
    return W_L2 * jnp.sum(parts[:, :L]) + W_KL * jnp.sum(parts[:, L:])


@jax.jit
def _loss(hp, hp_rot, labels):
    return _tc_loss(labels, hp, hp_rot) + _sc_loss(hp, hp_rot, labels)


def kernel(hp, hp_rot, label_rot):
    return _loss(hp, hp_rot, label_rot.astype(jnp.int32))
